# Initial kernel scaffold; baseline (speedup 1.0000x reference)
import functools

import jax
import jax.numpy as jnp
from jax import lax
from jax.experimental import pallas as pl
from jax.experimental.pallas import tpu as pltpu
from jax.experimental.pallas import tpu_sc as plsc

DIM = 64
B = 4096
L = 50
NW = 32
BPW = B // NW
LANES = 16

_mesh = plsc.VectorSubcoreMesh(core_axis_name="c", subcore_axis_name="s")


def _sc_body(dev_hbm, res_hbm, table_hbm, out_d, out_r,
             idx_d, idx_r, idxT_d, idxT_r, acc_d, acc_r, sem):
    wid = lax.axis_index("s") * 2 + lax.axis_index("c")
    base = wid * BPW

    pltpu.sync_copy(dev_hbm.at[pl.ds(base, BPW), :], idx_d)
    pltpu.sync_copy(res_hbm.at[pl.ds(base, BPW), :], idx_r)

    def transpose(src, dst):
        def body(l, carry):
            lvec = jnp.full((LANES,), l, dtype=jnp.int32)
            for bc in range(BPW // LANES):
                rows = bc * LANES + lax.iota(jnp.int32, LANES)
                v = plsc.load_gather(src, [rows, lvec])
                dst[l, pl.ds(bc * LANES, LANES)] = v
            return carry
        lax.fori_loop(0, L, body, 0)

    transpose(idx_d, idxT_d)
    transpose(idx_r, idxT_r)

    pltpu.async_copy(table_hbm.at[idxT_d.at[0]], acc_d, sem).wait()
    pltpu.async_copy(table_hbm.at[idxT_r.at[0]], acc_r, sem).wait()
    for l in range(1, L):
        pltpu.async_copy(table_hbm.at[idxT_d.at[l]], acc_d, sem, add=True)
        pltpu.async_copy(table_hbm.at[idxT_r.at[l]], acc_r, sem, add=True)
    for l in range(1, L):
        pltpu.make_async_copy(table_hbm.at[idxT_d.at[l]], acc_d, sem).wait()
        pltpu.make_async_copy(table_hbm.at[idxT_r.at[l]], acc_r, sem).wait()

    pltpu.sync_copy(acc_d, out_d.at[pl.ds(base, BPW), :])
    pltpu.sync_copy(acc_r, out_r.at[pl.ds(base, BPW), :])


_sc_gather = functools.partial(
    pl.kernel,
    out_type=[jax.ShapeDtypeStruct((B, DIM), jnp.float32),
              jax.ShapeDtypeStruct((B, DIM), jnp.float32)],
    mesh=_mesh,
    scratch_types=[
        pltpu.VMEM((BPW, L), jnp.int32),
        pltpu.VMEM((BPW, L), jnp.int32),
        pltpu.VMEM((L, BPW), jnp.int32),
        pltpu.VMEM((L, BPW), jnp.int32),
        pltpu.VMEM((BPW, DIM), jnp.float32),
        pltpu.VMEM((BPW, DIM), jnp.float32),
        pltpu.SemaphoreType.DMA,
    ],
)(_sc_body)


BS = 512


def _tc_body(td_ref, tr_ref, idd_ref, idr_ref, tbl_ref, w_ref, b_ref, out_ref):
    tbl = tbl_ref[...]

    def corrected(sums, ids):
        acc = sums
        for k in range(4):
            cnt = jnp.sum((ids == k).astype(jnp.float32), axis=1, keepdims=True)
            acc = acc - cnt * tbl[k][None, :]
        return acc

    text = corrected(td_ref[...], idd_ref[...])
    resv = corrected(tr_ref[...], idr_ref[...])
    w = w_ref[...]
    z = (jnp.dot(text, w[:DIM], preferred_element_type=jnp.float32)
         + jnp.dot(resv, w[DIM:], preferred_element_type=jnp.float32)
         + b_ref[...])
    out_ref[...] = jnp.tanh(z)


def _tc_proj(td, tr, dev, res, emb_table, w, b2d):
    return pl.pallas_call(
        _tc_body,
        grid=(B // BS,),
        in_specs=[
            pl.BlockSpec((BS, DIM), lambda i: (i, 0)),
            pl.BlockSpec((BS, DIM), lambda i: (i, 0)),
            pl.BlockSpec((BS, L), lambda i: (i, 0)),
            pl.BlockSpec((BS, L), lambda i: (i, 0)),
            pl.BlockSpec((8, DIM), lambda i: (0, 0)),
            pl.BlockSpec((2 * DIM, DIM), lambda i: (0, 0)),
            pl.BlockSpec((1, DIM), lambda i: (0, 0)),
        ],
        out_specs=pl.BlockSpec((BS, DIM), lambda i: (i, 0)),
        out_shape=jax.ShapeDtypeStruct((B, DIM), jnp.float32),
    )(td, tr, dev, res, emb_table, w, b2d)


def kernel(developer_token_id, resource_token_id, emb_table, W_proj, b_proj):
    dev = developer_token_id.astype(jnp.int32)
    res = resource_token_id.astype(jnp.int32)
    td, tr = _sc_gather(dev, res, emb_table)
    return _tc_proj(td, tr, dev, res, emb_table, W_proj,
                    b_proj.reshape(1, DIM))


# speedup vs baseline: 1.3346x; 1.3346x over previous
import functools

import jax
import jax.numpy as jnp
from jax import lax
from jax.experimental import pallas as pl
from jax.experimental.pallas import tpu as pltpu
from jax.experimental.pallas import tpu_sc as plsc

DIM = 64
B = 4096
L = 50
NW = 32
BPW = B // NW
LANES = 16

_mesh = plsc.VectorSubcoreMesh(core_axis_name="c", subcore_axis_name="s")


def _sc_body(dev_hbm, res_hbm, table_hbm, out_d, out_r,
             idx_d, idx_r, idxT_d, idxT_r, acc_d, acc_r, sem):
    wid = lax.axis_index("s") * 2 + lax.axis_index("c")
    base = wid * BPW

    pltpu.sync_copy(dev_hbm.at[pl.ds(base, BPW), :], idx_d)
    pltpu.sync_copy(res_hbm.at[pl.ds(base, BPW), :], idx_r)

    def transpose(src, dst):
        def body(l, carry):
            lvec = jnp.full((LANES,), l, dtype=jnp.int32)
            for bc in range(BPW // LANES):
                rows = bc * LANES + lax.iota(jnp.int32, LANES)
                v = plsc.load_gather(src, [rows, lvec])
                dst[l, pl.ds(bc * LANES, LANES)] = v
            return carry
        lax.fori_loop(0, L, body, 0)

    transpose(idx_d, idxT_d)
    transpose(idx_r, idxT_r)

    pltpu.async_copy(table_hbm.at[idxT_d.at[0]], acc_d, sem).wait()
    pltpu.async_copy(table_hbm.at[idxT_r.at[0]], acc_r, sem).wait()
    for l in range(1, L):
        pltpu.async_copy(table_hbm.at[idxT_d.at[l]], acc_d, sem, add=True)
        pltpu.async_copy(table_hbm.at[idxT_r.at[l]], acc_r, sem, add=True)
    for l in range(1, L):
        pltpu.make_async_copy(table_hbm.at[idxT_d.at[l]], acc_d, sem).wait()
        pltpu.make_async_copy(table_hbm.at[idxT_r.at[l]], acc_r, sem).wait()

    pltpu.sync_copy(acc_d, out_d.at[pl.ds(base, BPW), :])
    pltpu.sync_copy(acc_r, out_r.at[pl.ds(base, BPW), :])


_sc_gather = functools.partial(
    pl.kernel,
    out_type=[jax.ShapeDtypeStruct((B, DIM), jnp.float32),
              jax.ShapeDtypeStruct((B, DIM), jnp.float32)],
    mesh=_mesh,
    compiler_params=pltpu.CompilerParams(needs_layout_passes=False,
                                         use_tc_tiling_on_sc=False),
    scratch_types=[
        pltpu.VMEM((BPW, L), jnp.int32),
        pltpu.VMEM((BPW, L), jnp.int32),
        pltpu.VMEM((L, BPW), jnp.int32),
        pltpu.VMEM((L, BPW), jnp.int32),
        pltpu.VMEM((BPW, DIM), jnp.float32),
        pltpu.VMEM((BPW, DIM), jnp.float32),
        pltpu.SemaphoreType.DMA,
    ],
)(_sc_body)


BS = 512


def _tc_body(td_ref, tr_ref, idd_ref, idr_ref, tbl_ref, w_ref, b_ref, out_ref):
    tbl = tbl_ref[...]

    def corrected(sums, ids):
        acc = sums
        for k in range(4):
            cnt = jnp.sum((ids == k).astype(jnp.float32), axis=1, keepdims=True)
            acc = acc - cnt * tbl[k][None, :]
        return acc

    text = corrected(td_ref[...], idd_ref[...])
    resv = corrected(tr_ref[...], idr_ref[...])
    w = w_ref[...]
    z = (jnp.dot(text, w[:DIM], preferred_element_type=jnp.float32)
         + jnp.dot(resv, w[DIM:], preferred_element_type=jnp.float32)
         + b_ref[...])
    out_ref[...] = jnp.tanh(z)


def _tc_proj(td, tr, dev, res, emb_table, w, b2d):
    return pl.pallas_call(
        _tc_body,
        grid=(B // BS,),
        in_specs=[
            pl.BlockSpec((BS, DIM), lambda i: (i, 0)),
            pl.BlockSpec((BS, DIM), lambda i: (i, 0)),
            pl.BlockSpec((BS, L), lambda i: (i, 0)),
            pl.BlockSpec((BS, L), lambda i: (i, 0)),
            pl.BlockSpec((8, DIM), lambda i: (0, 0)),
            pl.BlockSpec((2 * DIM, DIM), lambda i: (0, 0)),
            pl.BlockSpec((1, DIM), lambda i: (0, 0)),
        ],
        out_specs=pl.BlockSpec((BS, DIM), lambda i: (i, 0)),
        out_shape=jax.ShapeDtypeStruct((B, DIM), jnp.float32),
    )(td, tr, dev, res, emb_table, w, b2d)


def kernel(developer_token_id, resource_token_id, emb_table, W_proj, b_proj):
    dev = developer_token_id.astype(jnp.int32)
    res = resource_token_id.astype(jnp.int32)
    td, tr = _sc_gather(dev, res, emb_table)
    return _tc_proj(td, tr, dev, res, emb_table, W_proj,
                    b_proj.reshape(1, DIM))
